# baseline (speedup 1.0000x reference)
import jax
import jax.numpy as jnp
from jax import lax
from jax.experimental import pallas as pl
from jax.experimental.pallas import tpu as pltpu, tpu_sc as plsc

_B, _T, _C = 2, 4096, 2048
_NC, _NS = 2, 16
_NW = _NC * _NS
_L = 16
_KB = _C // 128
_TRW = _KB * 8 * 128
_NTR = _B * _T // 8
_NCHUNK = _NTR // _NW
_NPAIR = _NCHUNK // 2
_NSLOT = 5


def _body(x_hbm, e0_hbm, e1_hbm, out_hbm,
          x_all, o0, o1, e0_v, e1_v, gsem, ssem):
    wid = lax.axis_index("s") * _NC + lax.axis_index("c")
    start_tr = wid * _NCHUNK
    is_bstart = (start_tr % (_T // 8)) == 0
    is_first_w = wid == 0

    pltpu.sync_copy(e0_hbm, e0_v)
    pltpu.sync_copy(e1_hbm, e1_v)

    @pl.when(jnp.logical_not(is_first_w))
    def _():
        pltpu.sync_copy(x_hbm.at[pl.ds((start_tr - 1) * _TRW, _TRW)],
                        x_all.at[pl.ds((_NSLOT - 1) * _TRW, _TRW)])

    pltpu.async_copy(x_hbm.at[pl.ds(start_tr * _TRW, _TRW)],
                     x_all.at[pl.ds(0, _TRW)], gsem)
    pltpu.async_copy(x_hbm.at[pl.ds((start_tr + 1) * _TRW, _TRW)],
                     x_all.at[pl.ds(_TRW, _TRW)], gsem)

    def pair_body(p, carry):
        t0 = start_tr + 2 * p
        sl0 = lax.rem(2 * p, _NSLOT) * _TRW
        sl1 = lax.rem(2 * p + 1, _NSLOT) * _TRW
        slp = lax.rem(2 * p + _NSLOT - 1, _NSLOT) * _TRW

        pltpu.make_async_copy(
            x_hbm.at[pl.ds(t0 * _TRW, _TRW)],
            x_all.at[pl.ds(sl0, _TRW)], gsem).wait()
        pltpu.make_async_copy(
            x_hbm.at[pl.ds((t0 + 1) * _TRW, _TRW)],
            x_all.at[pl.ds(sl1, _TRW)], gsem).wait()

        @pl.when(p + 1 < _NPAIR)
        def _():
            sn0 = lax.rem(2 * p + 2, _NSLOT) * _TRW
            sn1 = lax.rem(2 * p + 3, _NSLOT) * _TRW
            pltpu.async_copy(x_hbm.at[pl.ds((t0 + 2) * _TRW, _TRW)],
                             x_all.at[pl.ds(sn0, _TRW)], gsem)
            pltpu.async_copy(x_hbm.at[pl.ds((t0 + 3) * _TRW, _TRW)],
                             x_all.at[pl.ds(sn1, _TRW)], gsem)

        @pl.when(p >= 1)
        def _():
            pltpu.make_async_copy(
                o0, out_hbm.at[pl.ds((t0 - 2) * _TRW, _TRW)], ssem).wait()
            pltpu.make_async_copy(
                o1, out_hbm.at[pl.ds((t0 - 1) * _TRW, _TRW)], ssem).wait()

        def j_body(j, c):
            for gl in range(8):
                eb = j * 128 + gl * _L
                e0s = e0_v[pl.ds(eb, _L)]
                e1s = e1_v[pl.ds(eb, _L)]
                qb = j * 1024 + gl * _L
                xv = [x_all[pl.ds(slp + qb + 896, _L)]] + \
                     [x_all[pl.ds(sl0 + qb + s * 128 - 128, _L)]
                      for s in range(1, 8)] + \
                     [x_all[pl.ds(sl0 + qb + 896, _L)]] + \
                     [x_all[pl.ds(sl1 + qb + s * 128 - 128, _L)]
                      for s in range(1, 8)]
                for s in range(8):
                    o0[pl.ds(qb + s * 128, _L)] = \
                        jnp.where(xv[s] > 0, e1s, e0s)
                for s in range(8):
                    o1[pl.ds(qb + s * 128, _L)] = \
                        jnp.where(xv[8 + s] > 0, e1s, e0s)
            return c

        lax.fori_loop(0, _KB, j_body, 0)

        @pl.when(jnp.logical_and(is_bstart, p == 0))
        def _():
            def fix(j, c):
                def fixgl(gl, c):
                    o0[pl.ds(j * 1024 + gl * _L, _L)] = \
                        e0_v[pl.ds(j * 128 + gl * _L, _L)]
                    return c
                return lax.fori_loop(0, 8, fixgl, c)
            lax.fori_loop(0, _KB, fix, 0)

        pltpu.async_copy(o0, out_hbm.at[pl.ds(t0 * _TRW, _TRW)], ssem)
        pltpu.async_copy(o1, out_hbm.at[pl.ds((t0 + 1) * _TRW, _TRW)], ssem)
        return carry

    lax.fori_loop(0, _NPAIR, pair_body, 0)

    pltpu.make_async_copy(
        o0, out_hbm.at[pl.ds((start_tr + _NCHUNK - 2) * _TRW, _TRW)],
        ssem).wait()
    pltpu.make_async_copy(
        o1, out_hbm.at[pl.ds((start_tr + _NCHUNK - 1) * _TRW, _TRW)],
        ssem).wait()


@jax.jit
def kernel(x, emb0, emb1):
    B, T, C = x.shape
    x_flat = x.reshape(B * T // 8, 8, C // 128, 128) \
              .transpose(0, 2, 1, 3).reshape(-1)
    e0 = emb0.reshape(C)
    e1 = emb1.reshape(C)

    mesh = plsc.VectorSubcoreMesh(core_axis_name="c", subcore_axis_name="s",
                                  num_cores=_NC, num_subcores=_NS)
    out_flat = pl.kernel(
        _body,
        out_type=jax.ShapeDtypeStruct((B * T * C,), jnp.float32),
        mesh=mesh,
        scratch_types=[
            pltpu.VMEM((_NSLOT * _TRW,), jnp.float32),
            pltpu.VMEM((_TRW,), jnp.float32),
            pltpu.VMEM((_TRW,), jnp.float32),
            pltpu.VMEM((_C,), jnp.float32),
            pltpu.VMEM((_C,), jnp.float32),
            pltpu.SemaphoreType.DMA,
            pltpu.SemaphoreType.DMA,
        ],
    )(x_flat, e0, e1)
    return out_flat.reshape(B * T // 8, C // 128, 8, 128) \
                   .transpose(0, 2, 1, 3).reshape(B, T, C)


# speedup vs baseline: 1.1026x; 1.1026x over previous
import jax
import jax.numpy as jnp
from jax import lax
from jax.experimental import pallas as pl
from jax.experimental.pallas import tpu as pltpu, tpu_sc as plsc

_B, _T, _C = 2, 4096, 2048
_NC, _NS = 2, 16
_NW = _NC * _NS
_L = 16
_KB = _C // 128
_TRW = _KB * 8 * 128
_NTR = _B * _T // 8
_NCHUNK = _NTR // _NW


def _body(x_hbm, e0_hbm, e1_hbm, out_hbm,
          x0, x1, x2, x3, o0, o1, e0_v, e1_v,
          g0, g1, g2, g3, s0, s1):
    wid = lax.axis_index("s") * _NC + lax.axis_index("c")
    start_tr = wid * _NCHUNK
    is_bstart = (start_tr % (_T // 8)) == 0
    is_first_w = wid == 0

    xbufs = [x0, x1, x2, x3]
    obufs = [o0, o1]
    gsems = [g0, g1, g2, g3]
    ssems = [s0, s1]

    pltpu.sync_copy(e0_hbm, e0_v)
    pltpu.sync_copy(e1_hbm, e1_v)

    @pl.when(jnp.logical_not(is_first_w))
    def _():
        pltpu.sync_copy(x_hbm.at[pl.ds((start_tr - 1) * _TRW, _TRW)], x3)

    pltpu.async_copy(x_hbm.at[pl.ds(start_tr * _TRW, _TRW)], x0, g0)
    pltpu.async_copy(x_hbm.at[pl.ds((start_tr + 1) * _TRW, _TRW)], x1, g1)

    def super_body(it, carry):
        base = it * 4
        for u in range(4):
            ci = base + u
            cur = xbufs[u]
            prv = xbufs[(u - 1) % 4]
            ob = obufs[u % 2]

            pltpu.make_async_copy(
                x_hbm.at[pl.ds((start_tr + ci) * _TRW, _TRW)], cur,
                gsems[u]).wait()

            @pl.when(ci + 2 < _NCHUNK)
            def _(ci=ci, u=u):
                pltpu.async_copy(
                    x_hbm.at[pl.ds((start_tr + ci + 2) * _TRW, _TRW)],
                    xbufs[(u + 2) % 4], gsems[(u + 2) % 4])

            @pl.when(ci >= 2)
            def _(ci=ci, u=u, ob=ob):
                pltpu.make_async_copy(
                    ob,
                    out_hbm.at[pl.ds((start_tr + ci - 2) * _TRW, _TRW)],
                    ssems[u % 2]).wait()

            def j_body(j, c, cur=cur, prv=prv, ob=ob):
                for gl in range(8):
                    eb = j * 128 + gl * _L
                    e0s = e0_v[pl.ds(eb, _L)]
                    e1s = e1_v[pl.ds(eb, _L)]
                    qb = j * 1024 + gl * _L
                    xv = [prv[pl.ds(qb + 896, _L)]] + \
                         [cur[pl.ds(qb + s * 128 - 128, _L)]
                          for s in range(1, 8)]
                    for s in range(8):
                        ob[pl.ds(qb + s * 128, _L)] = \
                            jnp.where(xv[s] > 0, e1s, e0s)
                return c

            lax.fori_loop(0, _KB, j_body, 0, unroll=4)

            @pl.when(jnp.logical_and(is_bstart, ci == 0))
            def _(ob=ob):
                def fix(j, c):
                    def fixgl(gl, c):
                        ob[pl.ds(j * 1024 + gl * _L, _L)] = \
                            e0_v[pl.ds(j * 128 + gl * _L, _L)]
                        return c
                    return lax.fori_loop(0, 8, fixgl, c)
                lax.fori_loop(0, _KB, fix, 0)

            pltpu.async_copy(
                ob, out_hbm.at[pl.ds((start_tr + ci) * _TRW, _TRW)],
                ssems[u % 2])
        return carry

    lax.fori_loop(0, _NCHUNK // 4, super_body, 0)

    pltpu.make_async_copy(
        o0, out_hbm.at[pl.ds((start_tr + _NCHUNK - 2) * _TRW, _TRW)],
        s0).wait()
    pltpu.make_async_copy(
        o1, out_hbm.at[pl.ds((start_tr + _NCHUNK - 1) * _TRW, _TRW)],
        s1).wait()


@jax.jit
def kernel(x, emb0, emb1):
    B, T, C = x.shape
    x_flat = x.reshape(B * T // 8, 8, C // 128, 128) \
              .transpose(0, 2, 1, 3).reshape(-1)
    e0 = emb0.reshape(C)
    e1 = emb1.reshape(C)

    mesh = plsc.VectorSubcoreMesh(core_axis_name="c", subcore_axis_name="s",
                                  num_cores=_NC, num_subcores=_NS)
    out_flat = pl.kernel(
        _body,
        out_type=jax.ShapeDtypeStruct((B * T * C,), jnp.float32),
        mesh=mesh,
        scratch_types=[
            pltpu.VMEM((_TRW,), jnp.float32),
            pltpu.VMEM((_TRW,), jnp.float32),
            pltpu.VMEM((_TRW,), jnp.float32),
            pltpu.VMEM((_TRW,), jnp.float32),
            pltpu.VMEM((_TRW,), jnp.float32),
            pltpu.VMEM((_TRW,), jnp.float32),
            pltpu.VMEM((_C,), jnp.float32),
            pltpu.VMEM((_C,), jnp.float32),
            pltpu.SemaphoreType.DMA,
            pltpu.SemaphoreType.DMA,
            pltpu.SemaphoreType.DMA,
            pltpu.SemaphoreType.DMA,
            pltpu.SemaphoreType.DMA,
            pltpu.SemaphoreType.DMA,
        ],
    )(x_flat, e0, e1)
    return out_flat.reshape(B * T // 8, C // 128, 8, 128) \
                   .transpose(0, 2, 1, 3).reshape(B, T, C)


# speedup vs baseline: 1.1986x; 1.0871x over previous
import jax
import jax.numpy as jnp
from jax import lax
from jax.experimental import pallas as pl
from jax.experimental.pallas import tpu as pltpu, tpu_sc as plsc

_B, _T, _C = 2, 4096, 2048
_NC, _NS = 2, 16
_NW = _NC * _NS
_L = 16
_KB = _C // 128
_TRW = _KB * 8 * 128
_NTR = _B * _T // 8
_NCHUNK = _NTR // _NW


def _body(x_hbm, e0_hbm, e1_hbm, out_hbm,
          x0, x1, x2, x3, o0, o1, e0_v, e1_v,
          g0, g1, g2, g3, s0, s1):
    wid = lax.axis_index("s") * _NC + lax.axis_index("c")
    start_tr = wid * _NCHUNK
    is_bstart = (start_tr % (_T // 8)) == 0
    is_first_w = wid == 0

    xbufs = [x0, x1, x2, x3]
    obufs = [o0, o1]
    gsems = [g0, g1, g2, g3]
    ssems = [s0, s1]

    pltpu.async_copy(x_hbm.at[pl.ds(start_tr * _TRW, _TRW)], x0, g0)
    pltpu.async_copy(x_hbm.at[pl.ds((start_tr + 1) * _TRW, _TRW)], x1, g1)

    @pl.when(jnp.logical_not(is_first_w))
    def _():
        pltpu.async_copy(x_hbm.at[pl.ds((start_tr - 1) * _TRW, _TRW)], x3, g3)

    pltpu.async_copy(e0_hbm, e0_v, s0)
    pltpu.async_copy(e1_hbm, e1_v, s1)

    pltpu.make_async_copy(e0_hbm, e0_v, s0).wait()
    pltpu.make_async_copy(e1_hbm, e1_v, s1).wait()

    @pl.when(jnp.logical_not(is_first_w))
    def _():
        pltpu.make_async_copy(
            x_hbm.at[pl.ds((start_tr - 1) * _TRW, _TRW)], x3, g3).wait()

    def super_body(it, carry):
        base = it * 4
        for u in range(4):
            ci = base + u
            cur = xbufs[u]
            prv = xbufs[(u - 1) % 4]
            ob = obufs[u % 2]

            pltpu.make_async_copy(
                x_hbm.at[pl.ds((start_tr + ci) * _TRW, _TRW)], cur,
                gsems[u]).wait()

            @pl.when(ci + 2 < _NCHUNK)
            def _(ci=ci, u=u):
                pltpu.async_copy(
                    x_hbm.at[pl.ds((start_tr + ci + 2) * _TRW, _TRW)],
                    xbufs[(u + 2) % 4], gsems[(u + 2) % 4])

            @pl.when(ci >= 2)
            def _(ci=ci, u=u, ob=ob):
                pltpu.make_async_copy(
                    ob,
                    out_hbm.at[pl.ds((start_tr + ci - 2) * _TRW, _TRW)],
                    ssems[u % 2]).wait()

            def j_body(j, c, cur=cur, prv=prv, ob=ob):
                for gl in range(8):
                    eb = j * 128 + gl * _L
                    e0s = e0_v[pl.ds(eb, _L)]
                    e1s = e1_v[pl.ds(eb, _L)]
                    qb = j * 1024 + gl * _L
                    xv = [prv[pl.ds(qb + 896, _L)]] + \
                         [cur[pl.ds(qb + s * 128 - 128, _L)]
                          for s in range(1, 8)]
                    for s in range(8):
                        ob[pl.ds(qb + s * 128, _L)] = \
                            jnp.where(xv[s] > 0, e1s, e0s)
                return c

            lax.fori_loop(0, _KB, j_body, 0, unroll=2)

            @pl.when(jnp.logical_and(is_bstart, ci == 0))
            def _(ob=ob):
                def fix(j, c):
                    def fixgl(gl, c):
                        ob[pl.ds(j * 1024 + gl * _L, _L)] = \
                            e0_v[pl.ds(j * 128 + gl * _L, _L)]
                        return c
                    return lax.fori_loop(0, 8, fixgl, c)
                lax.fori_loop(0, _KB, fix, 0)

            pltpu.async_copy(
                ob, out_hbm.at[pl.ds((start_tr + ci) * _TRW, _TRW)],
                ssems[u % 2])
        return carry

    lax.fori_loop(0, _NCHUNK // 4, super_body, 0)

    pltpu.make_async_copy(
        o0, out_hbm.at[pl.ds((start_tr + _NCHUNK - 2) * _TRW, _TRW)],
        s0).wait()
    pltpu.make_async_copy(
        o1, out_hbm.at[pl.ds((start_tr + _NCHUNK - 1) * _TRW, _TRW)],
        s1).wait()


@jax.jit
def kernel(x, emb0, emb1):
    B, T, C = x.shape
    x_flat = x.reshape(B * T // 8, 8, C // 128, 128) \
              .transpose(0, 2, 1, 3).reshape(-1)
    e0 = emb0.reshape(C)
    e1 = emb1.reshape(C)

    mesh = plsc.VectorSubcoreMesh(core_axis_name="c", subcore_axis_name="s",
                                  num_cores=_NC, num_subcores=_NS)
    out_flat = pl.kernel(
        _body,
        out_type=jax.ShapeDtypeStruct((B * T * C,), jnp.float32),
        mesh=mesh,
        scratch_types=[
            pltpu.VMEM((_TRW,), jnp.float32),
            pltpu.VMEM((_TRW,), jnp.float32),
            pltpu.VMEM((_TRW,), jnp.float32),
            pltpu.VMEM((_TRW,), jnp.float32),
            pltpu.VMEM((_TRW,), jnp.float32),
            pltpu.VMEM((_TRW,), jnp.float32),
            pltpu.VMEM((_C,), jnp.float32),
            pltpu.VMEM((_C,), jnp.float32),
            pltpu.SemaphoreType.DMA,
            pltpu.SemaphoreType.DMA,
            pltpu.SemaphoreType.DMA,
            pltpu.SemaphoreType.DMA,
            pltpu.SemaphoreType.DMA,
            pltpu.SemaphoreType.DMA,
        ],
    )(x_flat, e0, e1)
    return out_flat.reshape(B * T // 8, C // 128, 8, 128) \
                   .transpose(0, 2, 1, 3).reshape(B, T, C)


# speedup vs baseline: 1.2218x; 1.0194x over previous
import jax
import jax.numpy as jnp
from jax import lax
from jax.experimental import pallas as pl
from jax.experimental.pallas import tpu as pltpu, tpu_sc as plsc

_B, _T, _C = 2, 4096, 2048
_NC, _NS = 2, 16
_NW = _NC * _NS
_L = 16
_KB = _C // 128
_TRW = _KB * 8 * 128
_NTR = _B * _T // 8
_NCHUNK = _NTR // _NW


def _body(x_hbm, e0_hbm, e1_hbm, out_hbm,
          x0, x1, x2, x3, o0, o1, e0_v, e1_v,
          g0, g1, g2, g3, s0, s1):
    wid = lax.axis_index("s") * _NC + lax.axis_index("c")
    start_tr = wid * _NCHUNK
    is_bstart = (start_tr % (_T // 8)) == 0
    is_first_w = wid == 0

    xbufs = [x0, x1, x2, x3]
    obufs = [o0, o1]
    gsems = [g0, g1, g2, g3]
    ssems = [s0, s1]

    pltpu.async_copy(x_hbm.at[pl.ds(start_tr * _TRW, _TRW)], x0, g0)
    pltpu.async_copy(x_hbm.at[pl.ds((start_tr + 1) * _TRW, _TRW)], x1, g1)

    @pl.when(jnp.logical_not(is_first_w))
    def _():
        pltpu.async_copy(x_hbm.at[pl.ds((start_tr - 1) * _TRW, _TRW)], x3, g3)

    pltpu.async_copy(e0_hbm, e0_v, s0)
    pltpu.async_copy(e1_hbm, e1_v, s1)

    pltpu.make_async_copy(e0_hbm, e0_v, s0).wait()
    pltpu.make_async_copy(e1_hbm, e1_v, s1).wait()

    @pl.when(jnp.logical_not(is_first_w))
    def _():
        pltpu.make_async_copy(
            x_hbm.at[pl.ds((start_tr - 1) * _TRW, _TRW)], x3, g3).wait()

    def super_body(it, carry):
        base = it * 4
        for u in range(4):
            ci = base + u
            cur = xbufs[u]
            prv = xbufs[(u - 1) % 4]
            ob = obufs[u % 2]

            pltpu.make_async_copy(
                x_hbm.at[pl.ds((start_tr + ci) * _TRW, _TRW)], cur,
                gsems[u]).wait()

            @pl.when(ci + 2 < _NCHUNK)
            def _(ci=ci, u=u):
                pltpu.async_copy(
                    x_hbm.at[pl.ds((start_tr + ci + 2) * _TRW, _TRW)],
                    xbufs[(u + 2) % 4], gsems[(u + 2) % 4])

            @pl.when(ci >= 2)
            def _(ci=ci, u=u, ob=ob):
                pltpu.make_async_copy(
                    ob,
                    out_hbm.at[pl.ds((start_tr + ci - 2) * _TRW, _TRW)],
                    ssems[u % 2]).wait()

            @plsc.parallel_loop(0, _KB, unroll=2)
            def j_body(j, cur=cur, prv=prv, ob=ob):
                for gl in range(8):
                    eb = j * 128 + gl * _L
                    e0s = e0_v[pl.ds(eb, _L)]
                    e1s = e1_v[pl.ds(eb, _L)]
                    qb = j * 1024 + gl * _L
                    xv = [prv[pl.ds(qb + 896, _L)]] + \
                         [cur[pl.ds(qb + s * 128 - 128, _L)]
                          for s in range(1, 8)]
                    for s in range(8):
                        ob[pl.ds(qb + s * 128, _L)] = \
                            jnp.where(xv[s] > 0, e1s, e0s)

            @pl.when(jnp.logical_and(is_bstart, ci == 0))
            def _(ob=ob):
                def fix(j, c):
                    def fixgl(gl, c):
                        ob[pl.ds(j * 1024 + gl * _L, _L)] = \
                            e0_v[pl.ds(j * 128 + gl * _L, _L)]
                        return c
                    return lax.fori_loop(0, 8, fixgl, c)
                lax.fori_loop(0, _KB, fix, 0)

            pltpu.async_copy(
                ob, out_hbm.at[pl.ds((start_tr + ci) * _TRW, _TRW)],
                ssems[u % 2])
        return carry

    lax.fori_loop(0, _NCHUNK // 4, super_body, 0)

    pltpu.make_async_copy(
        o0, out_hbm.at[pl.ds((start_tr + _NCHUNK - 2) * _TRW, _TRW)],
        s0).wait()
    pltpu.make_async_copy(
        o1, out_hbm.at[pl.ds((start_tr + _NCHUNK - 1) * _TRW, _TRW)],
        s1).wait()


@jax.jit
def kernel(x, emb0, emb1):
    B, T, C = x.shape
    x_flat = x.reshape(B * T // 8, 8, C // 128, 128) \
              .transpose(0, 2, 1, 3).reshape(-1)
    e0 = emb0.reshape(C)
    e1 = emb1.reshape(C)

    mesh = plsc.VectorSubcoreMesh(core_axis_name="c", subcore_axis_name="s",
                                  num_cores=_NC, num_subcores=_NS)
    out_flat = pl.kernel(
        _body,
        out_type=jax.ShapeDtypeStruct((B * T * C,), jnp.float32),
        mesh=mesh,
        scratch_types=[
            pltpu.VMEM((_TRW,), jnp.float32),
            pltpu.VMEM((_TRW,), jnp.float32),
            pltpu.VMEM((_TRW,), jnp.float32),
            pltpu.VMEM((_TRW,), jnp.float32),
            pltpu.VMEM((_TRW,), jnp.float32),
            pltpu.VMEM((_TRW,), jnp.float32),
            pltpu.VMEM((_C,), jnp.float32),
            pltpu.VMEM((_C,), jnp.float32),
            pltpu.SemaphoreType.DMA,
            pltpu.SemaphoreType.DMA,
            pltpu.SemaphoreType.DMA,
            pltpu.SemaphoreType.DMA,
            pltpu.SemaphoreType.DMA,
            pltpu.SemaphoreType.DMA,
        ],
    )(x_flat, e0, e1)
    return out_flat.reshape(B * T // 8, C // 128, 8, 128) \
                   .transpose(0, 2, 1, 3).reshape(B, T, C)
